# baseline (speedup 1.0000x reference)
import functools

import jax
import jax.numpy as jnp
from jax import lax
from jax.experimental import pallas as pl
from jax.experimental.pallas import tpu as pltpu
from jax.experimental.pallas import tpu_sc as plsc

N = 50000
H = 128
E = 150000

NS = 16
BLK = 2432
NBLK = 4
STRIP = BLK * NBLK
EPAD = NS * STRIP
NVB = BLK // 16
GCH = 64
NCB = BLK // GCH
CHUNK = 5120
NPASS = 5
ACCR = CHUNK + 8
DUMMY = CHUNK
TPB = CHUNK // NS
NPAD = 10 * CHUNK

_mesh = plsc.VectorSubcoreMesh(core_axis_name="c", subcore_axis_name="s")

_SCRATCH = [
    pltpu.VMEM((BLK,), jnp.int32),
    pltpu.VMEM((BLK,), jnp.int32),
    pltpu.VMEM((NCB, GCH), jnp.int32),
    pltpu.VMEM((GCH, H), jnp.float32),
    pltpu.VMEM((GCH, H), jnp.float32),
    pltpu.VMEM_SHARED((ACCR, H), jnp.float32),
    pltpu.SemaphoreType.DMA,
]


def _zero_rows(rows, zf16):
    def _zr(t, _):
        rows[t >> 3, pl.ds((t & 7) * 16, 16)] = zf16
        return 0
    lax.fori_loop(0, GCH * 8, _zr, 0)


def _zero_acc_slice(rows, acc, s):
    def _za(i, _):
        pltpu.sync_copy(rows, acc.at[pl.ds(s * TPB + i * GCH, GCH)])
        return 0
    lax.fori_loop(0, TPB // GCH, _za, 0)

    @pl.when(s == 0)
    def _zdum():
        pltpu.sync_copy(rows.at[pl.ds(0, 8)], acc.at[pl.ds(DUMMY, 8)])


def _compute_dloc(dstb, dloc, lo):
    def _pos(i, _):
        dv = dstb[pl.ds(i * 16, 16)]
        u = dv - lo
        mi = 1 + ((u >> 31) | ((CHUNK - 1 - u) >> 31))
        dloc[i >> 2, pl.ds((i & 3) * 16, 16)] = u * mi + DUMMY * (1 - mi)
        return 0
    lax.fori_loop(0, NVB, _pos, 0)


def _flush_acc(acc, rows, out, s, lo):
    def _fl(i, _):
        pltpu.sync_copy(acc.at[pl.ds(s * TPB + i * GCH, GCH)], rows)
        pltpu.sync_copy(rows, out.at[pl.ds(lo + s * TPB + i * GCH, GCH)])
        return 0
    lax.fori_loop(0, TPB // GCH, _fl, 0)


@functools.partial(
    pl.kernel,
    mesh=_mesh,
    out_type=jax.ShapeDtypeStruct((NPAD, H), jnp.float32),
    scratch_types=_SCRATCH,
)
def _sc_scatter_rel(xh, srch, dsth, s_out,
                    dstb, srcb, dloc, rows, ones, acc, sem):
    c = lax.axis_index("c")
    s = lax.axis_index("s")
    zf16 = jnp.zeros((16,), jnp.float32)

    def _do_pass(p, _):
        lo = (p * 2 + c) * CHUNK
        _zero_rows(rows, zf16)
        _zero_acc_slice(rows, acc, s)
        plsc.subcore_barrier()

        def _block(b, _):
            pltpu.sync_copy(dsth.at[pl.ds(s * STRIP + b * BLK, BLK)], dstb)
            pltpu.sync_copy(srch.at[pl.ds(s * STRIP + b * BLK, BLK)], srcb)
            _compute_dloc(dstb, dloc, lo)

            def _gs(j, _):
                pltpu.async_copy(xh.at[srcb.at[pl.ds(j * GCH, GCH)]],
                                 rows, sem).wait()
                pltpu.sync_copy(rows, acc.at[dloc.at[j]], add=True)
                return 0
            lax.fori_loop(0, NCB, _gs, 0)
            return 0
        lax.fori_loop(0, NBLK, _block, 0)
        plsc.subcore_barrier()
        _flush_acc(acc, rows, s_out, s, lo)
        plsc.subcore_barrier()
        return 0

    lax.fori_loop(0, NPASS, _do_pass, 0)


@functools.partial(
    pl.kernel,
    mesh=_mesh,
    out_type=jax.ShapeDtypeStruct((NPAD, H), jnp.float32),
    scratch_types=_SCRATCH,
)
def _sc_count_type(dsta, dstc, c_out,
                   dstb, srcb, dloc, rows, ones, acc, sem):
    c = lax.axis_index("c")
    s = lax.axis_index("s")
    lane = lax.iota(jnp.int32, 16)
    one0 = jnp.where(lane == 0, 1.0, 0.0).astype(jnp.float32)
    zf16 = jnp.zeros((16,), jnp.float32)

    def _fill_ones(t, _):
        k = t & 7
        ones[t >> 3, pl.ds(k * 16, 16)] = jnp.where(k == 0, one0, zf16)
        return 0
    lax.fori_loop(0, GCH * 8, _fill_ones, 0)

    def _do_pass(p, _):
        lo = (p * 2 + c) * CHUNK
        _zero_rows(rows, zf16)
        _zero_acc_slice(rows, acc, s)
        plsc.subcore_barrier()

        for dsth in (dsta, dstc):
            def _block(b, _, dsth=dsth):
                pltpu.sync_copy(dsth.at[pl.ds(s * STRIP + b * BLK, BLK)], dstb)
                _compute_dloc(dstb, dloc, lo)

                def _gs(j, _):
                    pltpu.sync_copy(ones, acc.at[dloc.at[j]], add=True)
                    return 0
                lax.fori_loop(0, NCB, _gs, 0)
                return 0
            lax.fori_loop(0, NBLK, _block, 0)
        plsc.subcore_barrier()
        _flush_acc(acc, rows, c_out, s, lo)
        plsc.subcore_barrier()
        return 0

    lax.fori_loop(0, NPASS, _do_pass, 0)


_BR = 512


def _dense_body(csm_ref, x_ref, sa_ref, sb_ref, cnt_ref, basis_ref,
                ws_ref, lw_ref, lb_ref, out_ref, *, rel_a, rel_b):
    v0 = basis_ref[0]
    v1 = basis_ref[1]
    w_a = csm_ref[rel_a, 0] * v0 + csm_ref[rel_a, 1] * v1
    w_b = csm_ref[rel_b, 0] * v0 + csm_ref[rel_b, 1] * v1
    x = x_ref[...]
    agg = (jnp.dot(sa_ref[...], w_a.T, preferred_element_type=jnp.float32)
           + jnp.dot(sb_ref[...], w_b.T, preferred_element_type=jnp.float32))
    deg = jnp.maximum(cnt_ref[:, :1], 1.0)
    h_self = jnp.dot(x, ws_ref[...].T, preferred_element_type=jnp.float32)
    h = jnp.maximum(h_self + agg / deg, 0.0) + x
    mu = jnp.mean(h, axis=-1, keepdims=True)
    d = h - mu
    var = jnp.mean(d * d, axis=-1, keepdims=True)
    out_ref[...] = d * lax.rsqrt(var + 1e-5) * lw_ref[...] + lb_ref[...]


def _dense_stage(csm, x, s_a, s_b, cnt, basis, w_self, lw, lb, rel_a, rel_b):
    grid = (NPAD // _BR,)
    row = lambda i: (i, 0)
    fixed = lambda i: (0, 0)
    out = pl.pallas_call(
        functools.partial(_dense_body, rel_a=rel_a, rel_b=rel_b),
        grid=grid,
        in_specs=[
            pl.BlockSpec(memory_space=pltpu.SMEM),
            pl.BlockSpec((_BR, H), row),
            pl.BlockSpec((_BR, H), row),
            pl.BlockSpec((_BR, H), row),
            pl.BlockSpec((_BR, H), row),
            pl.BlockSpec((2, H, H), lambda i: (0, 0, 0)),
            pl.BlockSpec((H, H), fixed),
            pl.BlockSpec((1, H), fixed),
            pl.BlockSpec((1, H), fixed),
        ],
        out_specs=pl.BlockSpec((_BR, H), row),
        out_shape=jax.ShapeDtypeStruct((NPAD, H), jnp.float32),
    )(csm, x, s_a, s_b, cnt, basis, w_self, lw, lb)
    return out[:N]


def _pad_ei(ei):
    src = jnp.concatenate([ei[0], jnp.zeros((EPAD - E,), jnp.int32)])
    dst = jnp.concatenate([ei[1], jnp.full((EPAD - E,), -1, jnp.int32)])
    return src, dst


def kernel(x_user, x_item, ei_rates, ei_rated_by, ei_follows, ei_similar,
           basis, coeff, W_self_user, W_self_item,
           ln_w_user, ln_b_user, ln_w_item, ln_b_item):
    csm = jax.nn.softmax(coeff, axis=-1)
    src0, dst0 = _pad_ei(ei_rates)
    src1, dst1 = _pad_ei(ei_rated_by)
    src2, dst2 = _pad_ei(ei_follows)
    src3, dst3 = _pad_ei(ei_similar)

    s_rates = _sc_scatter_rel(x_user, src0, dst0)
    s_ratedby = _sc_scatter_rel(x_item, src1, dst1)
    s_follows = _sc_scatter_rel(x_user, src2, dst2)
    s_similar = _sc_scatter_rel(x_item, src3, dst3)
    cnt_user = _sc_count_type(dst1, dst2)
    cnt_item = _sc_count_type(dst0, dst3)

    pad = lambda a: jnp.pad(a, ((0, NPAD - N), (0, 0)))
    out_user = _dense_stage(
        csm, pad(x_user), s_ratedby, s_follows, cnt_user,
        basis, W_self_user, ln_w_user[None, :], ln_b_user[None, :],
        rel_a=1, rel_b=2)
    out_item = _dense_stage(
        csm, pad(x_item), s_rates, s_similar, cnt_item,
        basis, W_self_item, ln_w_item[None, :], ln_b_item[None, :],
        rel_a=0, rel_b=3)
    return (out_user, out_item)


# speedup vs baseline: 1.0296x; 1.0296x over previous
import functools

import jax
import jax.numpy as jnp
from jax import lax
from jax.experimental import pallas as pl
from jax.experimental.pallas import tpu as pltpu
from jax.experimental.pallas import tpu_sc as plsc

N = 50000
H = 128
E = 150000

NS = 16
BLK = 2432
NBLK = 4
STRIP = BLK * NBLK
EPAD = NS * STRIP
NVB = BLK // 16
GCH = 64
NCB = BLK // GCH
CHUNK = 5120
NPASS = 5
ACCR = CHUNK + 8
DUMMY = CHUNK
TPB = CHUNK // NS
NPAD = 10 * CHUNK

_mesh = plsc.VectorSubcoreMesh(core_axis_name="c", subcore_axis_name="s")

_SCRATCH = [
    pltpu.VMEM((BLK,), jnp.int32),
    pltpu.VMEM((BLK,), jnp.int32),
    pltpu.VMEM((NCB, GCH), jnp.int32),
    pltpu.VMEM((GCH, H), jnp.float32),
    pltpu.VMEM((GCH, H), jnp.float32),
    pltpu.VMEM_SHARED((ACCR, H), jnp.float32),
    pltpu.SemaphoreType.DMA,
]


def _zero_rows(rows, zf16):
    def _zr(t, _):
        rows[t >> 3, pl.ds((t & 7) * 16, 16)] = zf16
        return 0
    lax.fori_loop(0, GCH * 8, _zr, 0)


def _zero_acc_slice(rows, acc, s):
    def _za(i, _):
        pltpu.sync_copy(rows, acc.at[pl.ds(s * TPB + i * GCH, GCH)])
        return 0
    lax.fori_loop(0, TPB // GCH, _za, 0)

    @pl.when(s == 0)
    def _zdum():
        pltpu.sync_copy(rows.at[pl.ds(0, 8)], acc.at[pl.ds(DUMMY, 8)])


def _compute_dloc(dstb, dloc, lo):
    def _pos(i, _):
        dv = dstb[pl.ds(i * 16, 16)]
        u = dv - lo
        mi = 1 + ((u >> 31) | ((CHUNK - 1 - u) >> 31))
        dloc[i >> 2, pl.ds((i & 3) * 16, 16)] = u * mi + DUMMY * (1 - mi)
        return 0
    lax.fori_loop(0, NVB, _pos, 0)


def _flush_acc(acc, rows, out, s, lo):
    def _fl(i, _):
        pltpu.sync_copy(acc.at[pl.ds(s * TPB + i * GCH, GCH)], rows)
        pltpu.sync_copy(rows, out.at[pl.ds(lo + s * TPB + i * GCH, GCH)])
        return 0
    lax.fori_loop(0, TPB // GCH, _fl, 0)


DCH = 128
NCD = BLK // DCH

_SCRATCH_F = [
    pltpu.VMEM((BLK,), jnp.int32),
    pltpu.VMEM((BLK,), jnp.int32),
    pltpu.VMEM((NCD, DCH), jnp.int32),
    pltpu.VMEM((DCH, H), jnp.float32),
    pltpu.VMEM((DCH, H), jnp.float32),
    pltpu.VMEM_SHARED((ACCR, H), jnp.float32),
    pltpu.SemaphoreType.DMA,
    pltpu.SemaphoreType.DMA,
]


@functools.partial(
    pl.kernel,
    mesh=_mesh,
    out_type=jax.ShapeDtypeStruct((NPAD, H), jnp.float32),
    scratch_types=_SCRATCH_F,
)
def _sc_scatter_rel(xh, srch, dsth, s_out,
                    dstb, srcb, dloc, rowsa, rowsb, acc, sema, semb):
    c = lax.axis_index("c")
    s = lax.axis_index("s")
    zf16 = jnp.zeros((16,), jnp.float32)
    bufs = (rowsa, rowsb)
    sems = (sema, semb)

    def _do_pass(p, _):
        lo = (p * 2 + c) * CHUNK

        def _zr(t, _):
            rowsa[t >> 3, pl.ds((t & 7) * 16, 16)] = zf16
            return 0
        lax.fori_loop(0, DCH * 8, _zr, 0)
        def _za(i, _):
            pltpu.sync_copy(rowsa, acc.at[pl.ds(s * TPB + i * DCH, DCH)])
            return 0
        lax.fori_loop(0, 2, _za, 0)
        pltpu.sync_copy(rowsa.at[pl.ds(0, 64)],
                        acc.at[pl.ds(s * TPB + 256, 64)])

        @pl.when(s == 0)
        def _zdum():
            pltpu.sync_copy(rowsa.at[pl.ds(0, 8)], acc.at[pl.ds(DUMMY, 8)])
        plsc.subcore_barrier()

        def _block(b, _):
            pltpu.sync_copy(dsth.at[pl.ds(s * STRIP + b * BLK, BLK)], dstb)
            pltpu.sync_copy(srch.at[pl.ds(s * STRIP + b * BLK, BLK)], srcb)

            def _pos(i, _):
                dv = dstb[pl.ds(i * 16, 16)]
                u = dv - lo
                mi = 1 + ((u >> 31) | ((CHUNK - 1 - u) >> 31))
                dloc[i >> 3, pl.ds((i & 7) * 16, 16)] = (
                    u * mi + DUMMY * (1 - mi))
                return 0
            lax.fori_loop(0, NVB, _pos, 0)

            cps = [None, None]
            cps[0] = pltpu.async_copy(
                xh.at[srcb.at[pl.ds(0, DCH)]], bufs[0], sems[0])
            for j in range(NCD):
                if j + 1 < NCD:
                    cps[(j + 1) % 2] = pltpu.async_copy(
                        xh.at[srcb.at[pl.ds((j + 1) * DCH, DCH)]],
                        bufs[(j + 1) % 2], sems[(j + 1) % 2])
                cps[j % 2].wait()
                pltpu.sync_copy(bufs[j % 2], acc.at[dloc.at[j]], add=True)
            return 0
        lax.fori_loop(0, NBLK, _block, 0)
        plsc.subcore_barrier()
        def _fl(i, _):
            pltpu.sync_copy(acc.at[pl.ds(s * TPB + i * DCH, DCH)], rowsa)
            pltpu.sync_copy(rowsa,
                            s_out.at[pl.ds(lo + s * TPB + i * DCH, DCH)])
            return 0
        lax.fori_loop(0, 2, _fl, 0)
        pltpu.sync_copy(acc.at[pl.ds(s * TPB + 256, 64)],
                        rowsa.at[pl.ds(0, 64)])
        pltpu.sync_copy(rowsa.at[pl.ds(0, 64)],
                        s_out.at[pl.ds(lo + s * TPB + 256, 64)])
        plsc.subcore_barrier()
        return 0

    lax.fori_loop(0, NPASS, _do_pass, 0)


@functools.partial(
    pl.kernel,
    mesh=_mesh,
    out_type=jax.ShapeDtypeStruct((NPAD, H), jnp.float32),
    scratch_types=_SCRATCH,
)
def _sc_count_type(dsta, dstc, c_out,
                   dstb, srcb, dloc, rows, ones, acc, sem):
    c = lax.axis_index("c")
    s = lax.axis_index("s")
    lane = lax.iota(jnp.int32, 16)
    one0 = jnp.where(lane == 0, 1.0, 0.0).astype(jnp.float32)
    zf16 = jnp.zeros((16,), jnp.float32)

    def _fill_ones(t, _):
        k = t & 7
        ones[t >> 3, pl.ds(k * 16, 16)] = jnp.where(k == 0, one0, zf16)
        return 0
    lax.fori_loop(0, GCH * 8, _fill_ones, 0)

    def _do_pass(p, _):
        lo = (p * 2 + c) * CHUNK
        _zero_rows(rows, zf16)
        _zero_acc_slice(rows, acc, s)
        plsc.subcore_barrier()

        for dsth in (dsta, dstc):
            def _block(b, _, dsth=dsth):
                pltpu.sync_copy(dsth.at[pl.ds(s * STRIP + b * BLK, BLK)], dstb)
                _compute_dloc(dstb, dloc, lo)

                def _gs(j, _):
                    pltpu.sync_copy(ones, acc.at[dloc.at[j]], add=True)
                    return 0
                lax.fori_loop(0, NCB, _gs, 0)
                return 0
            lax.fori_loop(0, NBLK, _block, 0)
        plsc.subcore_barrier()
        _flush_acc(acc, rows, c_out, s, lo)
        plsc.subcore_barrier()
        return 0

    lax.fori_loop(0, NPASS, _do_pass, 0)


_BR = 512


def _dense_body(csm_ref, x_ref, sa_ref, sb_ref, cnt_ref, basis_ref,
                ws_ref, lw_ref, lb_ref, out_ref, *, rel_a, rel_b):
    v0 = basis_ref[0]
    v1 = basis_ref[1]
    w_a = csm_ref[rel_a, 0] * v0 + csm_ref[rel_a, 1] * v1
    w_b = csm_ref[rel_b, 0] * v0 + csm_ref[rel_b, 1] * v1
    x = x_ref[...]
    agg = (jnp.dot(sa_ref[...], w_a.T, preferred_element_type=jnp.float32)
           + jnp.dot(sb_ref[...], w_b.T, preferred_element_type=jnp.float32))
    deg = jnp.maximum(cnt_ref[:, :1], 1.0)
    h_self = jnp.dot(x, ws_ref[...].T, preferred_element_type=jnp.float32)
    h = jnp.maximum(h_self + agg / deg, 0.0) + x
    mu = jnp.mean(h, axis=-1, keepdims=True)
    d = h - mu
    var = jnp.mean(d * d, axis=-1, keepdims=True)
    out_ref[...] = d * lax.rsqrt(var + 1e-5) * lw_ref[...] + lb_ref[...]


def _dense_stage(csm, x, s_a, s_b, cnt, basis, w_self, lw, lb, rel_a, rel_b):
    grid = (NPAD // _BR,)
    row = lambda i: (i, 0)
    fixed = lambda i: (0, 0)
    out = pl.pallas_call(
        functools.partial(_dense_body, rel_a=rel_a, rel_b=rel_b),
        grid=grid,
        in_specs=[
            pl.BlockSpec(memory_space=pltpu.SMEM),
            pl.BlockSpec((_BR, H), row),
            pl.BlockSpec((_BR, H), row),
            pl.BlockSpec((_BR, H), row),
            pl.BlockSpec((_BR, H), row),
            pl.BlockSpec((2, H, H), lambda i: (0, 0, 0)),
            pl.BlockSpec((H, H), fixed),
            pl.BlockSpec((1, H), fixed),
            pl.BlockSpec((1, H), fixed),
        ],
        out_specs=pl.BlockSpec((_BR, H), row),
        out_shape=jax.ShapeDtypeStruct((NPAD, H), jnp.float32),
    )(csm, x, s_a, s_b, cnt, basis, w_self, lw, lb)
    return out[:N]


def _pad_ei(ei):
    src = jnp.concatenate([ei[0], jnp.zeros((EPAD - E,), jnp.int32)])
    dst = jnp.concatenate([ei[1], jnp.full((EPAD - E,), -1, jnp.int32)])
    return src, dst


def kernel(x_user, x_item, ei_rates, ei_rated_by, ei_follows, ei_similar,
           basis, coeff, W_self_user, W_self_item,
           ln_w_user, ln_b_user, ln_w_item, ln_b_item):
    csm = jax.nn.softmax(coeff, axis=-1)
    src0, dst0 = _pad_ei(ei_rates)
    src1, dst1 = _pad_ei(ei_rated_by)
    src2, dst2 = _pad_ei(ei_follows)
    src3, dst3 = _pad_ei(ei_similar)

    s_rates = _sc_scatter_rel(x_user, src0, dst0)
    s_ratedby = _sc_scatter_rel(x_item, src1, dst1)
    s_follows = _sc_scatter_rel(x_user, src2, dst2)
    s_similar = _sc_scatter_rel(x_item, src3, dst3)
    cnt_user = _sc_count_type(dst1, dst2)
    cnt_item = _sc_count_type(dst0, dst3)

    pad = lambda a: jnp.pad(a, ((0, NPAD - N), (0, 0)))
    out_user = _dense_stage(
        csm, pad(x_user), s_ratedby, s_follows, cnt_user,
        basis, W_self_user, ln_w_user[None, :], ln_b_user[None, :],
        rel_a=1, rel_b=2)
    out_item = _dense_stage(
        csm, pad(x_item), s_rates, s_similar, cnt_item,
        basis, W_self_item, ln_w_item[None, :], ln_b_item[None, :],
        rel_a=0, rel_b=3)
    return (out_user, out_item)
